# baseline (speedup 1.0000x reference)
import functools

import jax
import jax.numpy as jnp
from jax import lax
from jax.experimental import pallas as pl
from jax.experimental.pallas import tpu as pltpu
from jax.experimental.pallas import tpu_sc as plsc

_SHAPE = (96, 96)
_N = _SHAPE[0] * _SHAPE[1]
_NWORDS = _N // 4
_BETA = 0.9
_L = 16
_NC = 1
_NS = 16
_NW = _NC * _NS
_TW = 128
_NT = _N // _TW
_TPW = (_NT + _NW - 1) // _NW
_ROWS = 16


def _sc_body(spk_hbm, w_hbm, x_hbm, a_hbm, t_hbm, out_hbm,
             spk_v, wrd_v, idx_v, rows_v, acc_v, x_v, a_v, t_v, o_v,
             sem, sem_ops):
    cid = lax.axis_index("c")
    sid = lax.axis_index("s")
    wid = sid * _NC + cid

    pltpu.sync_copy(spk_hbm, spk_v)

    lanes = lax.iota(jnp.int32, _L)
    zi = jnp.zeros((_L,), jnp.int32)

    def or_body(i, o):
        return o | spk_v[pl.ds(i * _L, _L)]

    orv = lax.fori_loop(0, _NWORDS // _L, or_body, zi)
    any_spike = jnp.max(orv)

    def compact():
        def l1(i, nwvec):
            w = spk_v[pl.ds(i * _L, _L)]
            m = w != 0
            pos = nwvec + lax.cumsum(m.astype(jnp.int32)) - 1
            plsc.store_scatter(wrd_v, [pos], lanes + i * _L, mask=m)
            return nwvec + plsc.all_reduce_population_count(m)

        nwvec = lax.fori_loop(0, _NWORDS // _L, l1, zi)
        nw = jnp.max(nwvec)
        plsc.store_scatter(wrd_v, [nw + lanes], zi)

        def l2(j, kvec):
            widx = wrd_v[pl.ds(j * _L, _L)]
            w = plsc.load_gather(spk_v, [widx])
            valid = (j * _L + lanes) < nw
            k2 = kvec
            for b in range(4):
                vb = (w >> (8 * b)) & 0xFF
                mb = (vb != 0) & valid
                pos = k2 + lax.cumsum(mb.astype(jnp.int32)) - 1
                plsc.store_scatter(idx_v, [pos], widx * 4 + b, mask=mb)
                k2 = k2 + plsc.all_reduce_population_count(mb)
            return k2

        nbw = (nw + _L - 1) // _L
        kvec = lax.fori_loop(0, nbw, l2, zi)
        k = jnp.max(kvec)
        plsc.store_scatter(idx_v, [k + lanes], zi)
        return k

    k_total = lax.cond(any_spike != 0, compact, lambda: jnp.int32(0))

    nfull = k_total // _ROWS
    tail = k_total - nfull * _ROWS
    zf = jnp.zeros((_L,), jnp.float32)

    for t in range(_TPW):
        tile = wid + _NW * t

        @pl.when(tile < _NT)
        def _():
            c0 = pl.multiple_of(tile * _TW, _TW)

            cx = pltpu.async_copy(x_hbm.at[pl.ds(c0, _TW)], x_v, sem_ops)
            ca = pltpu.async_copy(a_hbm.at[pl.ds(c0, _TW)], a_v, sem_ops)
            ct = pltpu.async_copy(t_hbm.at[pl.ds(c0, _TW)], t_v, sem_ops)

            for i in range(_TW // _L):
                acc_v[pl.ds(i * _L, _L)] = zf

            def batch_body(b, carry):
                src = w_hbm.at[idx_v.at[pl.ds(b * _ROWS, _ROWS)],
                               pl.ds(c0, _TW)]
                pltpu.async_copy(src, rows_v, sem).wait()
                for r in range(_ROWS):
                    for ci in range(_TW // _L):
                        d = pl.ds(ci * _L, _L)
                        acc_v[d] = acc_v[d] + rows_v[r, d]
                return carry

            lax.fori_loop(0, nfull, batch_body, 0)

            @pl.when(tail > 0)
            def _():
                src = w_hbm.at[idx_v.at[pl.ds(nfull * _ROWS, _ROWS)],
                               pl.ds(c0, _TW)]
                pltpu.async_copy(src, rows_v, sem).wait()
                for r in range(_ROWS - 1):
                    @pl.when(r < tail)
                    def _():
                        for ci in range(_TW // _L):
                            d = pl.ds(ci * _L, _L)
                            acc_v[d] = acc_v[d] + rows_v[r, d]

            cx.wait()
            ca.wait()
            ct.wait()
            for ci in range(_TW // _L):
                d = pl.ds(ci * _L, _L)
                act = _BETA * a_v[d] + x_v[d] + acc_v[d]
                o_v[d] = (act > t_v[d]).astype(jnp.float32)
            pltpu.sync_copy(o_v, out_hbm.at[pl.ds(c0, _TW)])


@functools.cache
def _sc_step():
    return pl.kernel(
        _sc_body,
        out_type=jax.ShapeDtypeStruct((_N,), jnp.float32),
        mesh=plsc.VectorSubcoreMesh(core_axis_name="c", subcore_axis_name="s",
                                    num_cores=_NC, num_subcores=_NS),
        compiler_params=pltpu.CompilerParams(needs_layout_passes=False),
        scratch_types=[
            pltpu.VMEM((_NWORDS,), jnp.int32),
            pltpu.VMEM((_NWORDS + _L,), jnp.int32),
            pltpu.VMEM((_N + _L,), jnp.int32),
            pltpu.VMEM((_ROWS, _TW), jnp.float32),
            pltpu.VMEM((_TW,), jnp.float32),
            pltpu.VMEM((_TW,), jnp.float32),
            pltpu.VMEM((_TW,), jnp.float32),
            pltpu.VMEM((_TW,), jnp.float32),
            pltpu.VMEM((_TW,), jnp.float32),
            pltpu.SemaphoreType.DMA,
            pltpu.SemaphoreType.DMA,
        ],
    )


def kernel(x, lateral_weights, activation, spikes, threshold, freq_act):
    del freq_act
    s = spikes.reshape(_NWORDS, 4).astype(jnp.int32)
    spk = s[:, 0] | (s[:, 1] << 8) | (s[:, 2] << 16) | (s[:, 3] << 24)
    out = _sc_step()(spk, lateral_weights, x.reshape(-1),
                     activation.reshape(-1), threshold.reshape(-1))
    return out.reshape(_SHAPE).astype(bool)


# speedup vs baseline: 1.0743x; 1.0743x over previous
import functools

import jax
import jax.numpy as jnp
from jax import lax
from jax.experimental import pallas as pl
from jax.experimental.pallas import tpu as pltpu
from jax.experimental.pallas import tpu_sc as plsc

_SHAPE = (96, 96)
_N = _SHAPE[0] * _SHAPE[1]
_NWORDS = _N // 4
_BETA = 0.9
_L = 16
_NC = 2
_NS = 16
_NW = _NC * _NS
_TW = 128
_NT = _N // _TW
_TPW = (_NT + _NW - 1) // _NW
_ROWS = 16
_OR_UNROLL = 8


def _sc_body(spk_hbm, w_hbm, x_hbm, a_hbm, t_hbm, out_hbm,
             spk_v, wrd_v, idx_v, rows_v, acc_v, x_v, a_v, t_v, o_v,
             sem, sem_ops, sem_out):
    cid = lax.axis_index("c")
    sid = lax.axis_index("s")
    wid = sid * _NC + cid

    pltpu.sync_copy(spk_hbm, spk_v)

    for t in range(_TPW):
        tile = wid + _NW * t

        @pl.when(tile < _NT)
        def _():
            c0 = pl.multiple_of((wid + _NW * t) * _TW, _TW)
            pltpu.async_copy(x_hbm.at[pl.ds(c0, _TW)], x_v.at[t], sem_ops)
            pltpu.async_copy(a_hbm.at[pl.ds(c0, _TW)], a_v.at[t], sem_ops)
            pltpu.async_copy(t_hbm.at[pl.ds(c0, _TW)], t_v.at[t], sem_ops)

    lanes = lax.iota(jnp.int32, _L)
    zi = jnp.zeros((_L,), jnp.int32)
    zf = jnp.zeros((_L,), jnp.float32)

    for t in range(_TPW):
        for i in range(_TW // _L):
            acc_v[t, pl.ds(i * _L, _L)] = zf

    def or_body(j, o):
        for u in range(_OR_UNROLL):
            o = o | spk_v[pl.ds((j * _OR_UNROLL + u) * _L, _L)]
        return o

    orv = lax.fori_loop(0, _NWORDS // _L // _OR_UNROLL, or_body, zi)
    any_spike = jnp.max(orv)

    def compact():
        def l1(i, nwvec):
            w = spk_v[pl.ds(i * _L, _L)]
            m = w != 0
            pos = nwvec + lax.cumsum(m.astype(jnp.int32)) - 1
            plsc.store_scatter(wrd_v, [pos], lanes + i * _L, mask=m)
            return nwvec + plsc.all_reduce_population_count(m)

        nwvec = lax.fori_loop(0, _NWORDS // _L, l1, zi)
        nw = jnp.max(nwvec)
        plsc.store_scatter(wrd_v, [nw + lanes], zi)

        def l2(j, kvec):
            widx = wrd_v[pl.ds(j * _L, _L)]
            w = plsc.load_gather(spk_v, [widx])
            valid = (j * _L + lanes) < nw
            k2 = kvec
            for b in range(4):
                vb = (w >> (8 * b)) & 0xFF
                mb = (vb != 0) & valid
                pos = k2 + lax.cumsum(mb.astype(jnp.int32)) - 1
                plsc.store_scatter(idx_v, [pos], widx * 4 + b, mask=mb)
                k2 = k2 + plsc.all_reduce_population_count(mb)
            return k2

        nbw = (nw + _L - 1) // _L
        kvec = lax.fori_loop(0, nbw, l2, zi)
        k = jnp.max(kvec)
        plsc.store_scatter(idx_v, [k + lanes], zi)
        return k

    k_total = lax.cond(any_spike != 0, compact, lambda: jnp.int32(0))

    nfull = k_total // _ROWS
    tail = k_total - nfull * _ROWS

    @pl.when(k_total > 0)
    def _():
        for t in range(_TPW):
            tile = wid + _NW * t

            @pl.when(tile < _NT)
            def _():
                c0 = pl.multiple_of((wid + _NW * t) * _TW, _TW)

                def batch_body(b, carry):
                    src = w_hbm.at[idx_v.at[pl.ds(b * _ROWS, _ROWS)],
                                   pl.ds(c0, _TW)]
                    pltpu.async_copy(src, rows_v, sem).wait()
                    for r in range(_ROWS):
                        for ci in range(_TW // _L):
                            d = pl.ds(ci * _L, _L)
                            acc_v[t, d] = acc_v[t, d] + rows_v[r, d]
                    return carry

                lax.fori_loop(0, nfull, batch_body, 0)

                @pl.when(tail > 0)
                def _():
                    src = w_hbm.at[idx_v.at[pl.ds(nfull * _ROWS, _ROWS)],
                                   pl.ds(c0, _TW)]
                    pltpu.async_copy(src, rows_v, sem).wait()
                    for r in range(_ROWS - 1):
                        @pl.when(r < tail)
                        def _():
                            for ci in range(_TW // _L):
                                d = pl.ds(ci * _L, _L)
                                acc_v[t, d] = acc_v[t, d] + rows_v[r, d]

    for t in range(_TPW):
        tile = wid + _NW * t

        @pl.when(tile < _NT)
        def _():
            c0 = pl.multiple_of((wid + _NW * t) * _TW, _TW)
            pltpu.make_async_copy(x_hbm.at[pl.ds(c0, _TW)], x_v.at[t],
                                  sem_ops).wait()
            pltpu.make_async_copy(a_hbm.at[pl.ds(c0, _TW)], a_v.at[t],
                                  sem_ops).wait()
            pltpu.make_async_copy(t_hbm.at[pl.ds(c0, _TW)], t_v.at[t],
                                  sem_ops).wait()
            for ci in range(_TW // _L):
                d = pl.ds(ci * _L, _L)
                act = _BETA * a_v[t, d] + x_v[t, d] + acc_v[t, d]
                o_v[t, d] = (act > t_v[t, d]).astype(jnp.float32)
            pltpu.async_copy(o_v.at[t], out_hbm.at[pl.ds(c0, _TW)],
                             sem_out)

    for t in range(_TPW):
        tile = wid + _NW * t

        @pl.when(tile < _NT)
        def _():
            c0 = pl.multiple_of((wid + _NW * t) * _TW, _TW)
            pltpu.make_async_copy(o_v.at[t], out_hbm.at[pl.ds(c0, _TW)],
                                  sem_out).wait()


@functools.cache
def _sc_step():
    return pl.kernel(
        _sc_body,
        out_type=jax.ShapeDtypeStruct((_N,), jnp.float32),
        mesh=plsc.VectorSubcoreMesh(core_axis_name="c", subcore_axis_name="s",
                                    num_cores=_NC, num_subcores=_NS),
        compiler_params=pltpu.CompilerParams(needs_layout_passes=False),
        scratch_types=[
            pltpu.VMEM((_NWORDS,), jnp.int32),
            pltpu.VMEM((_NWORDS + _L,), jnp.int32),
            pltpu.VMEM((_N + _L,), jnp.int32),
            pltpu.VMEM((_ROWS, _TW), jnp.float32),
            pltpu.VMEM((_TPW, _TW), jnp.float32),
            pltpu.VMEM((_TPW, _TW), jnp.float32),
            pltpu.VMEM((_TPW, _TW), jnp.float32),
            pltpu.VMEM((_TPW, _TW), jnp.float32),
            pltpu.VMEM((_TPW, _TW), jnp.float32),
            pltpu.SemaphoreType.DMA,
            pltpu.SemaphoreType.DMA,
            pltpu.SemaphoreType.DMA,
        ],
    )


def kernel(x, lateral_weights, activation, spikes, threshold, freq_act):
    del freq_act
    s = spikes.reshape(_NWORDS, 4).astype(jnp.int32)
    spk = s[:, 0] | (s[:, 1] << 8) | (s[:, 2] << 16) | (s[:, 3] << 24)
    out = _sc_step()(spk, lateral_weights, x.reshape(-1),
                     activation.reshape(-1), threshold.reshape(-1))
    return out.reshape(_SHAPE).astype(bool)


# speedup vs baseline: 1.2475x; 1.1612x over previous
import functools

import jax
import jax.numpy as jnp
from jax import lax
from jax.experimental import pallas as pl
from jax.experimental.pallas import tpu as pltpu
from jax.experimental.pallas import tpu_sc as plsc

_SHAPE = (96, 96)
_N = _SHAPE[0] * _SHAPE[1]
_NWORDS = _N // 4
_BETA = 0.9
_L = 16
_NC = 2
_NS = 16
_BROWS = 8
_NWORK = _SHAPE[0] // _BROWS
_BW = _BROWS * _SHAPE[1]
_ROWS = 16
_OR_UNROLL = 8


def _sc_body(spk_hbm, w_hbm, x_hbm, a_hbm, t_hbm, out_hbm,
             spk_v, wrd_v, idx_v, rows_v, acc_v, x_v, a_v, t_v, o_v,
             sem, sem_ops):
    cid = lax.axis_index("c")
    sid = lax.axis_index("s")
    wid = sid * _NC + cid

    @pl.when(wid < _NWORK)
    def _():
        r0 = pl.multiple_of(wid * _BROWS, _BROWS)
        f0 = pl.multiple_of(wid * _BW, _BW)

        cx = pltpu.async_copy(x_hbm.at[pl.ds(r0, _BROWS), :], x_v, sem_ops)
        ca = pltpu.async_copy(a_hbm.at[pl.ds(r0, _BROWS), :], a_v, sem_ops)
        ct = pltpu.async_copy(t_hbm.at[pl.ds(r0, _BROWS), :], t_v, sem_ops)

        pltpu.sync_copy(spk_hbm, spk_v)

        lanes = lax.iota(jnp.int32, _L)
        zi = jnp.zeros((_L,), jnp.int32)
        zf = jnp.zeros((_L,), jnp.float32)

        for i in range(_BW // _L):
            acc_v[pl.ds(i * _L, _L)] = zf

        def or_body(j, o):
            for u in range(_OR_UNROLL):
                o = o | spk_v[pl.ds((j * _OR_UNROLL + u) * _L, _L)]
            return o

        orv = lax.fori_loop(0, _NWORDS // _L // _OR_UNROLL, or_body, zi)
        any_spike = jnp.max(orv)

        def compact():
            def l1(i, nwvec):
                w = spk_v[pl.ds(i * _L, _L)]
                m = w != 0
                pos = nwvec + lax.cumsum(m.astype(jnp.int32)) - 1
                plsc.store_scatter(wrd_v, [pos], lanes + i * _L, mask=m)
                return nwvec + plsc.all_reduce_population_count(m)

            nwvec = lax.fori_loop(0, _NWORDS // _L, l1, zi)
            nw = jnp.max(nwvec)
            plsc.store_scatter(wrd_v, [nw + lanes], zi)

            def l2(j, kvec):
                widx = wrd_v[pl.ds(j * _L, _L)]
                w = plsc.load_gather(spk_v, [widx])
                valid = (j * _L + lanes) < nw
                k2 = kvec
                for b in range(4):
                    vb = (w >> (8 * b)) & 0xFF
                    mb = (vb != 0) & valid
                    pos = k2 + lax.cumsum(mb.astype(jnp.int32)) - 1
                    plsc.store_scatter(idx_v, [pos], widx * 4 + b, mask=mb)
                    k2 = k2 + plsc.all_reduce_population_count(mb)
                return k2

            nbw = (nw + _L - 1) // _L
            kvec = lax.fori_loop(0, nbw, l2, zi)
            k = jnp.max(kvec)
            plsc.store_scatter(idx_v, [k + lanes], zi)
            return k

        k_total = lax.cond(any_spike != 0, compact, lambda: jnp.int32(0))

        nb = (k_total + _ROWS - 1) // _ROWS

        def batch_body(b, carry):
            src = w_hbm.at[idx_v.at[pl.ds(b * _ROWS, _ROWS)],
                           pl.ds(f0, _BW)]
            pltpu.async_copy(src, rows_v, sem).wait()
            nrows = jnp.minimum(_ROWS, k_total - b * _ROWS)

            def row_body(r, c2):
                for ci in range(_BW // _L):
                    d = pl.ds(ci * _L, _L)
                    acc_v[d] = acc_v[d] + rows_v[r, d]
                return c2

            lax.fori_loop(0, nrows, row_body, 0)
            return carry

        lax.fori_loop(0, nb, batch_body, 0)

        cx.wait()
        ca.wait()
        ct.wait()
        for rl in range(_BROWS):
            for c in range(_SHAPE[1] // _L):
                d = pl.ds(c * _L, _L)
                da = pl.ds(rl * _SHAPE[1] + c * _L, _L)
                act = _BETA * a_v[rl, d] + x_v[rl, d] + acc_v[da]
                o_v[rl, d] = (act > t_v[rl, d]).astype(jnp.float32)
        pltpu.sync_copy(o_v, out_hbm.at[pl.ds(r0, _BROWS), :])


@functools.cache
def _sc_step():
    return pl.kernel(
        _sc_body,
        out_type=jax.ShapeDtypeStruct(_SHAPE, jnp.float32),
        mesh=plsc.VectorSubcoreMesh(core_axis_name="c", subcore_axis_name="s",
                                    num_cores=_NC, num_subcores=_NS),
        compiler_params=pltpu.CompilerParams(needs_layout_passes=False),
        scratch_types=[
            pltpu.VMEM((_NWORDS,), jnp.int32),
            pltpu.VMEM((_NWORDS + _L,), jnp.int32),
            pltpu.VMEM((_N + _L,), jnp.int32),
            pltpu.VMEM((_ROWS, _BW), jnp.float32),
            pltpu.VMEM((_BW,), jnp.float32),
            pltpu.VMEM((_BROWS, _SHAPE[1]), jnp.float32),
            pltpu.VMEM((_BROWS, _SHAPE[1]), jnp.float32),
            pltpu.VMEM((_BROWS, _SHAPE[1]), jnp.float32),
            pltpu.VMEM((_BROWS, _SHAPE[1]), jnp.float32),
            pltpu.SemaphoreType.DMA,
            pltpu.SemaphoreType.DMA,
        ],
    )


def kernel(x, lateral_weights, activation, spikes, threshold, freq_act):
    del freq_act
    s = spikes.astype(jnp.int32)
    spk = (s[:, 0::4] | (s[:, 1::4] << 8)
           | (s[:, 2::4] << 16) | (s[:, 3::4] << 24)).reshape(-1)
    out = _sc_step()(spk, lateral_weights, x, activation, threshold)
    return out.astype(bool)
